# baseline (speedup 1.0000x reference)
import functools

import jax
import jax.numpy as jnp
from jax import lax
from jax.experimental import pallas as pl
from jax.experimental.pallas import tpu as pltpu
from jax.experimental.pallas import tpu_sc as plsc

N_NODES = 10000
N_EDGES = 320000
D = 128

NC = 2
NS = 16
NW = NC * NS

CHUNK = 64
CPS = 32
NSLAB = 5
NBUF = 4
EDGES_PER_TILE = NSLAB * CPS * CHUNK
E_PAD = NW * EDGES_PER_TILE
N_PAD = 10112
ROWS_PER_TILE = N_PAD // NS

_mesh = plsc.VectorSubcoreMesh(core_axis_name="c", subcore_axis_name="s")


@functools.partial(
    pl.kernel,
    mesh=_mesh,
    out_type=jax.ShapeDtypeStruct((NC, N_PAD, D), jnp.float32),
    scratch_types=[
        pltpu.VMEM((2, CPS, CHUNK), jnp.int32),
        pltpu.VMEM((2, CPS, CHUNK), jnp.int32),
        pltpu.VMEM((NBUF, CHUNK, D), jnp.float32),
        pltpu.VMEM_SHARED((N_PAD, D), jnp.float32),
        [pltpu.SemaphoreType.DMA] * NBUF,
        [pltpu.SemaphoreType.DMA] * NBUF,
        [pltpu.SemaphoreType.DMA] * 2,
        [pltpu.SemaphoreType.DMA] * 2,
    ],
)
def _sc_aggregate(feat_hbm, src_hbm, dst_hbm, zeros_hbm, part_hbm,
                  srcsl, dstsl, rows_v, acc_sh, sg, ss, sls, sld):
    c = lax.axis_index("c")
    s = lax.axis_index("s")
    wid = c * NS + s

    pltpu.async_copy(src_hbm.at[wid].at[0], srcsl.at[0], sls[0])
    pltpu.async_copy(dst_hbm.at[wid].at[0], dstsl.at[0], sld[0])
    pltpu.sync_copy(zeros_hbm.at[pl.ds(s * ROWS_PER_TILE, ROWS_PER_TILE)],
                    acc_sh.at[pl.ds(s * ROWS_PER_TILE, ROWS_PER_TILE)])
    plsc.subcore_barrier()

    def gather(p, j, b):
        return pltpu.make_async_copy(feat_hbm.at[srcsl.at[p].at[j]],
                                     rows_v.at[b], sg[b])

    def scat(p, j, b):
        return pltpu.make_async_copy(rows_v.at[b],
                                     acc_sh.at[dstsl.at[p].at[j]], ss[b])

    def issue_s(p, j, b):
        pltpu.async_copy(rows_v.at[b], acc_sh.at[dstsl.at[p].at[j]], ss[b],
                         add=True)

    for t in range(NSLAB):
        p = t % 2
        pltpu.make_async_copy(src_hbm.at[wid].at[t], srcsl.at[p],
                              sls[p]).wait()
        pltpu.make_async_copy(dst_hbm.at[wid].at[t], dstsl.at[p],
                              sld[p]).wait()
        if t + 1 < NSLAB:
            pn = (t + 1) % 2
            pltpu.async_copy(src_hbm.at[wid].at[t + 1], srcsl.at[pn], sls[pn])
            pltpu.async_copy(dst_hbm.at[wid].at[t + 1], dstsl.at[pn], sld[pn])

        gather(p, 0, 0).start()
        gather(p, 1, 1).start()
        for j in (0, 1):
            gather(p, j, j).wait()
            issue_s(p, j, j)
            gather(p, j + 2, j + 2).start()

        def mid(k, _):
            for m in range(4):
                j = 2 + k * 4 + m
                b = (2 + m) % 4
                gather(p, j, b).wait()
                issue_s(p, j, b)
                scat(p, j - 2, m).wait()
                gather(p, j + 2, m).start()
            return 0

        lax.fori_loop(0, (CPS - 4) // 4, mid, 0)

        for j in (CPS - 2, CPS - 1):
            b = j % 4
            gather(p, j, b).wait()
            issue_s(p, j, b)
            scat(p, j - 2, (j - 2) % 4).wait()
        scat(p, CPS - 2, (CPS - 2) % 4).wait()
        scat(p, CPS - 1, (CPS - 1) % 4).wait()

    plsc.subcore_barrier()
    pltpu.sync_copy(acc_sh.at[pl.ds(s * ROWS_PER_TILE, ROWS_PER_TILE)],
                    part_hbm.at[c].at[pl.ds(s * ROWS_PER_TILE, ROWS_PER_TILE)])


_ROW_BLK = 1000


def _tc_body(p_ref, wt_ref, b_ref, o_ref):
    agg = p_ref[0] + p_ref[1]
    o_ref[...] = (jnp.dot(agg, wt_ref[...], preferred_element_type=jnp.float32)
                  + b_ref[...])


def _tc_linear(partials, wt, b2):
    return pl.pallas_call(
        _tc_body,
        grid=(N_NODES // _ROW_BLK,),
        in_specs=[
            pl.BlockSpec((NC, _ROW_BLK, D), lambda i: (0, i, 0)),
            pl.BlockSpec((D, D), lambda i: (0, 0)),
            pl.BlockSpec((1, D), lambda i: (0, 0)),
        ],
        out_specs=pl.BlockSpec((_ROW_BLK, D), lambda i: (i, 0)),
        out_shape=jax.ShapeDtypeStruct((N_NODES, D), jnp.float32),
    )(partials, wt, b2)


def kernel(features, edge_index, W, b):
    src = edge_index[0].astype(jnp.int32)
    dst = edge_index[1].astype(jnp.int32)
    pad = E_PAD - N_EDGES
    src = jnp.concatenate([src, jnp.zeros((pad,), jnp.int32)])
    dst = jnp.concatenate([dst, jnp.full((pad,), N_NODES, jnp.int32)])
    src = src.reshape(NW, NSLAB, CPS, CHUNK)
    dst = dst.reshape(NW, NSLAB, CPS, CHUNK)
    zeros = jnp.zeros((N_PAD, D), jnp.float32)
    partials = _sc_aggregate(features, src, dst, zeros)
    return _tc_linear(partials, W.T, b.reshape(1, D))


# speedup vs baseline: 2.8554x; 2.8554x over previous
import functools

import jax
import jax.numpy as jnp
from jax import lax
from jax.experimental import pallas as pl
from jax.experimental.pallas import tpu as pltpu
from jax.experimental.pallas import tpu_sc as plsc

N_NODES = 10000
N_EDGES = 320000
D = 128

NC = 2
NS = 16
NW = NC * NS

EDGES_PER_TILE = N_EDGES // NW
CHUNK = 80
NCHUNK = EDGES_PER_TILE // CHUNK
N_PAD = 10240
ROWS_PER_TILE = N_PAD // NS

_mesh = plsc.VectorSubcoreMesh(core_axis_name="c", subcore_axis_name="s")


@functools.partial(
    pl.kernel,
    mesh=_mesh,
    out_type=jax.ShapeDtypeStruct((NC, N_PAD, D), jnp.float32),
    scratch_types=[
        pltpu.VMEM((EDGES_PER_TILE,), jnp.int32),
        pltpu.VMEM((NCHUNK, CHUNK), jnp.int32),
        pltpu.VMEM((2, CHUNK, D), jnp.float32),
        pltpu.VMEM_SHARED((N_PAD, D), jnp.float32),
        pltpu.SemaphoreType.DMA,
        pltpu.SemaphoreType.DMA,
        pltpu.SemaphoreType.DMA,
        pltpu.SemaphoreType.DMA,
    ],
)
def _sc_aggregate(feat_hbm, src_hbm, dst_hbm, zeros_hbm, part_hbm,
                  src_v, dst_v, rows_v, acc_sh, sg0, sg1, ss0, ss1):
    c = lax.axis_index("c")
    s = lax.axis_index("s")
    wid = c * NS + s
    sg = (sg0, sg1)
    ss = (ss0, ss1)

    pltpu.sync_copy(zeros_hbm.at[pl.ds(s * ROWS_PER_TILE, ROWS_PER_TILE)],
                    acc_sh.at[pl.ds(s * ROWS_PER_TILE, ROWS_PER_TILE)])

    pltpu.sync_copy(src_hbm.at[wid], src_v)
    pltpu.sync_copy(dst_hbm.at[wid], dst_v)
    plsc.subcore_barrier()

    def src_idx(i):
        return src_v.at[pl.ds(i * CHUNK, CHUNK)]

    def wait_g(i, b):
        pltpu.make_async_copy(feat_hbm.at[src_idx(i)], rows_v.at[b],
                              sg[b]).wait()

    def issue_s(i, b):
        pltpu.async_copy(rows_v.at[b], acc_sh.at[dst_v.at[i]], ss[b],
                         add=True)

    def wait_s(i, b):
        pltpu.make_async_copy(rows_v.at[b], acc_sh.at[dst_v.at[i]],
                              ss[b]).wait()

    def issue_g(i, b):
        pltpu.async_copy(feat_hbm.at[src_idx(i)], rows_v.at[b], sg[b])

    def full(i, b):
        wait_g(i, b)
        issue_s(i, b)
        wait_s(i - 1, 1 - b)
        issue_g(i + 1, 1 - b)

    issue_g(0, 0)
    wait_g(0, 0)
    issue_s(0, 0)
    issue_g(1, 1)

    def body(it, _):
        for m in range(2):
            full(1 + it * 2 + m, (1 + m) % 2)
        return 0

    lax.fori_loop(0, (NCHUNK - 3) // 2, body, 0)
    full(NCHUNK - 2, (NCHUNK - 2) % 2)
    i = NCHUNK - 1
    wait_g(i, i % 2)
    issue_s(i, i % 2)
    wait_s(i - 1, (i - 1) % 2)
    wait_s(i, i % 2)

    plsc.subcore_barrier()
    pltpu.sync_copy(acc_sh.at[pl.ds(s * ROWS_PER_TILE, ROWS_PER_TILE)],
                    part_hbm.at[c].at[pl.ds(s * ROWS_PER_TILE, ROWS_PER_TILE)])


_ROW_BLK = 1000


def _tc_body(p_ref, wt_ref, b_ref, o_ref):
    agg = p_ref[0] + p_ref[1]
    o_ref[...] = (jnp.dot(agg, wt_ref[...], preferred_element_type=jnp.float32)
                  + b_ref[...])


def _tc_linear(partials, wt, b2):
    return pl.pallas_call(
        _tc_body,
        grid=(N_NODES // _ROW_BLK,),
        in_specs=[
            pl.BlockSpec((NC, _ROW_BLK, D), lambda i: (0, i, 0)),
            pl.BlockSpec((D, D), lambda i: (0, 0)),
            pl.BlockSpec((1, D), lambda i: (0, 0)),
        ],
        out_specs=pl.BlockSpec((_ROW_BLK, D), lambda i: (i, 0)),
        out_shape=jax.ShapeDtypeStruct((N_NODES, D), jnp.float32),
    )(partials, wt, b2)


def kernel(features, edge_index, W, b):
    src = edge_index[0].astype(jnp.int32).reshape(NW, EDGES_PER_TILE)
    dst = edge_index[1].astype(jnp.int32).reshape(NW, NCHUNK, CHUNK)
    zeros = jnp.zeros((N_PAD, D), jnp.float32)
    partials = _sc_aggregate(features, src, dst, zeros)
    return _tc_linear(partials, W.T, b.reshape(1, D))


# speedup vs baseline: 4.0261x; 1.4100x over previous
import functools

import jax
import jax.numpy as jnp
from jax import lax
from jax.experimental import pallas as pl
from jax.experimental.pallas import tpu as pltpu
from jax.experimental.pallas import tpu_sc as plsc

N_NODES = 10000
N_EDGES = 320000
D = 128

NC = 2
NS = 16
NW = NC * NS

EDGES_PER_TILE = N_EDGES // NW
CHUNK = 80
NCHUNK = EDGES_PER_TILE // CHUNK
DSTAGE = 64
N_PAD = 10112
ROWS_PER_TILE = N_PAD // NS

_mesh = plsc.VectorSubcoreMesh(core_axis_name="c", subcore_axis_name="s")


@functools.partial(
    pl.kernel,
    mesh=_mesh,
    out_type=jax.ShapeDtypeStruct((NC, N_PAD, D), jnp.float32),
    scratch_types=[
        pltpu.VMEM((EDGES_PER_TILE,), jnp.int32),
        pltpu.VMEM((DSTAGE, CHUNK), jnp.int32),
        pltpu.VMEM((3, CHUNK, D), jnp.float32),
        pltpu.VMEM_SHARED((N_PAD, D), jnp.float32),
        [pltpu.SemaphoreType.DMA] * 3,
        [pltpu.SemaphoreType.DMA] * 3,
    ],
)
def _sc_aggregate(feat_hbm, src_hbm, dst_hbm, zeros_hbm, part_hbm,
                  src_v, dst_v, rows_v, acc_sh, sg, ss):
    c = lax.axis_index("c")
    s = lax.axis_index("s")
    wid = c * NS + s

    pltpu.sync_copy(zeros_hbm.at[pl.ds(s * ROWS_PER_TILE, ROWS_PER_TILE)],
                    acc_sh.at[pl.ds(s * ROWS_PER_TILE, ROWS_PER_TILE)])

    pltpu.sync_copy(src_hbm.at[wid], src_v)
    pltpu.sync_copy(dst_hbm.at[wid].at[pl.ds(0, DSTAGE)], dst_v)
    plsc.subcore_barrier()

    def src_idx(i):
        return src_v.at[pl.ds(i * CHUNK, CHUNK)]

    def issue_g(i, b):
        pltpu.async_copy(feat_hbm.at[src_idx(i)], rows_v.at[b], sg[b])

    def wait_g(i, b):
        pltpu.make_async_copy(feat_hbm.at[src_idx(i)], rows_v.at[b],
                              sg[b]).wait()

    def issue_s(row, b):
        pltpu.async_copy(rows_v.at[b], acc_sh.at[dst_v.at[row]], ss[b],
                         add=True)

    def wait_s(row, b):
        pltpu.make_async_copy(rows_v.at[b], acc_sh.at[dst_v.at[row]],
                              ss[b]).wait()

    def full(i, b, row):
        wait_g(i, b)
        issue_s(row, b)
        wait_s(row, b)
        issue_g(i + 3, b)

    def tail(i, b, row):
        wait_g(i, b)
        issue_s(row, b)
        wait_s(row, b)

    for b in range(3):
        issue_g(b, b)

    def body_a(it, _):
        for m in range(3):
            i = it * 3 + m
            full(i, m, i)
        return 0

    lax.fori_loop(0, (DSTAGE - 1) // 3, body_a, 0)
    full(DSTAGE - 1, (DSTAGE - 1) % 3, DSTAGE - 1)

    pltpu.sync_copy(dst_hbm.at[wid].at[pl.ds(DSTAGE, NCHUNK - DSTAGE)],
                    dst_v.at[pl.ds(0, NCHUNK - DSTAGE)])

    def body_b(it, _):
        for m in range(3):
            i = DSTAGE + it * 3 + m
            full(i, (DSTAGE + m) % 3, i - DSTAGE)
        return 0

    lax.fori_loop(0, 19, body_b, 0)
    full(NCHUNK - 4, (NCHUNK - 4) % 3, NCHUNK - 4 - DSTAGE)
    for i in range(NCHUNK - 3, NCHUNK):
        tail(i, i % 3, i - DSTAGE)

    plsc.subcore_barrier()
    pltpu.sync_copy(acc_sh.at[pl.ds(s * ROWS_PER_TILE, ROWS_PER_TILE)],
                    part_hbm.at[c].at[pl.ds(s * ROWS_PER_TILE, ROWS_PER_TILE)])


_ROW_BLK = 1000


def _tc_body(p_ref, wt_ref, b_ref, o_ref):
    agg = p_ref[0] + p_ref[1]
    o_ref[...] = (jnp.dot(agg, wt_ref[...], preferred_element_type=jnp.float32)
                  + b_ref[...])


def _tc_linear(partials, wt, b2):
    return pl.pallas_call(
        _tc_body,
        grid=(N_NODES // _ROW_BLK,),
        in_specs=[
            pl.BlockSpec((NC, _ROW_BLK, D), lambda i: (0, i, 0)),
            pl.BlockSpec((D, D), lambda i: (0, 0)),
            pl.BlockSpec((1, D), lambda i: (0, 0)),
        ],
        out_specs=pl.BlockSpec((_ROW_BLK, D), lambda i: (i, 0)),
        out_shape=jax.ShapeDtypeStruct((N_NODES, D), jnp.float32),
    )(partials, wt, b2)


def kernel(features, edge_index, W, b):
    src = edge_index[0].astype(jnp.int32).reshape(NW, EDGES_PER_TILE)
    dst = edge_index[1].astype(jnp.int32).reshape(NW, NCHUNK, CHUNK)
    zeros = jnp.zeros((N_PAD, D), jnp.float32)
    partials = _sc_aggregate(features, src, dst, zeros)
    return _tc_linear(partials, W.T, b.reshape(1, D))
